# Initial kernel scaffold; baseline (speedup 1.0000x reference)
import jax
import jax.numpy as jnp
from jax.experimental import pallas as pl


def kernel(x, base_W, base_b, router_W, A, Bm):
    raise NotImplementedError("write your pallas kernel here")


# speedup vs baseline: 21.4340x; 21.4340x over previous
import functools

import jax
import jax.numpy as jnp
from jax.experimental import pallas as pl

E = 8
TOP_K = 2
RANK = 8
ALPHA = 16.0
D_IN = 2048
D_OUT = 2048
ER = E * RANK

_BLK = 512


def _body(x_ref, w_ref, b_ref, rw_ref, a_ref, bm_ref, o_ref):
    xb = x_ref[...]
    dn_t = (((1,), (1,)), ((), ()))
    dn_n = (((1,), (0,)), ((), ()))

    y = jax.lax.dot_general(xb, w_ref[...], dn_t,
                            preferred_element_type=jnp.float32)
    y = y + b_ref[...]

    logits = jax.lax.dot_general(xb, rw_ref[...], dn_t,
                                 preferred_element_type=jnp.float32)
    iota = jax.lax.broadcasted_iota(jnp.int32, (_BLK, E), 1)
    v1 = jnp.max(logits, axis=-1, keepdims=True)
    i1 = jnp.min(jnp.where(logits == v1, iota, E), axis=-1, keepdims=True)
    masked = jnp.where(iota == i1, -jnp.inf, logits)
    v2 = jnp.max(masked, axis=-1, keepdims=True)
    i2 = jnp.min(jnp.where(masked == v2, iota, E), axis=-1, keepdims=True)
    ee = jnp.exp(v2 - v1)
    denom = 1.0 + ee
    p1 = (ALPHA / RANK) / denom
    p2 = (ALPHA / RANK) * ee / denom

    col = jax.lax.broadcasted_iota(jnp.int32, (_BLK, ER), 1) // RANK
    gmask = jnp.where(col == i1, p1, 0.0) + jnp.where(col == i2, p2, 0.0)

    h = jax.lax.dot_general(xb, a_ref[...], dn_t,
                            preferred_element_type=jnp.float32)
    fused = jax.lax.dot_general(h * gmask, bm_ref[...], dn_n,
                                preferred_element_type=jnp.float32)

    o_ref[...] = y + fused


@jax.jit
def kernel(x, base_W, base_b, router_W, A, Bm):
    b, s, _ = x.shape
    n = b * s
    x2 = x.reshape(n, D_IN)
    a_flat = A.reshape(ER, D_IN)
    bm_flat = jnp.transpose(Bm, (0, 2, 1)).reshape(ER, D_OUT)
    bias = base_b.reshape(1, D_OUT)

    grid = (n // _BLK,)
    out = pl.pallas_call(
        _body,
        grid=grid,
        in_specs=[
            pl.BlockSpec((_BLK, D_IN), lambda i: (i, 0)),
            pl.BlockSpec((D_OUT, D_IN), lambda i: (0, 0)),
            pl.BlockSpec((1, D_OUT), lambda i: (0, 0)),
            pl.BlockSpec((E, D_IN), lambda i: (0, 0)),
            pl.BlockSpec((ER, D_IN), lambda i: (0, 0)),
            pl.BlockSpec((ER, D_OUT), lambda i: (0, 0)),
        ],
        out_specs=pl.BlockSpec((_BLK, D_OUT), lambda i: (i, 0)),
        out_shape=jax.ShapeDtypeStruct((n, D_OUT), jnp.float32),
    )(x2, base_W, bias, router_W, a_flat, bm_flat)
    return out.reshape(b, s, D_OUT)
